# baseline (speedup 1.0000x reference)
import functools

import jax
import jax.numpy as jnp
from jax import lax
from jax.experimental import pallas as pl
from jax.experimental.pallas import tpu as pltpu
from jax.experimental.pallas import tpu_sc as plsc

VOCAB = 100000
DIM1 = 64
DIM2 = 128
DIM = DIM1 + DIM2
BATCH = 4096
SEQ = 200

NUM_CORES = 2
NUM_SUBCORES = 16
NW = NUM_CORES * NUM_SUBCORES
ROWS_PER_W = BATCH // NW

H0 = 104
H1 = SEQ - H0
HSIZE = (H0, H1)

RG = 4
NGRP = ROWS_PER_W // RG
TBODY = NGRP // 2
LANES = 16


def _make_kernel():
    mesh = plsc.VectorSubcoreMesh(core_axis_name="c", subcore_axis_name="s")

    @functools.partial(
        pl.kernel,
        mesh=mesh,
        out_type=jax.ShapeDtypeStruct((BATCH, SEQ, DIM), jnp.float32),
        scratch_types=[
            pltpu.VMEM((RG, H0), jnp.int32),
            pltpu.VMEM((RG, H0), jnp.int32),
            pltpu.VMEM((RG, H1), jnp.int32),
            pltpu.VMEM((RG, H1), jnp.int32),
            pltpu.VMEM((H0, DIM), jnp.float32),
            pltpu.VMEM((H1, DIM), jnp.float32),
            pltpu.VMEM((H0, DIM2), jnp.float32),
            pltpu.VMEM((H1, DIM2), jnp.float32),
            pltpu.SemaphoreType.DMA,
            pltpu.SemaphoreType.DMA,
            pltpu.SemaphoreType.DMA,
            pltpu.SemaphoreType.DMA,
        ],
    )
    def stack_embed(
        wga_hbm,
        wgb_hbm,
        ta_hbm,
        t2_hbm,
        out_hbm,
        idxa0,
        idxa1,
        idxb0,
        idxb1,
        comb0,
        comb1,
        r20,
        r21,
        sem_g0,
        sem_g1,
        sem_i0,
        sem_i1,
    ):
        wid = lax.axis_index("s") * NUM_CORES + lax.axis_index("c")
        base_row = wid * ROWS_PER_W
        idxa = (idxa0, idxa1)
        idxb = (idxb0, idxb1)
        comb = (comb0, comb1)
        r2 = (r20, r21)
        sem_g = (sem_g0, sem_g1)
        sem_i = (sem_i0, sem_i1)

        def idx_row(p, r, h):
            return idxa[p].at[r] if h == 0 else idxb[p].at[r]

        def issue(p, r, h):
            pltpu.async_copy(
                ta_hbm.at[idx_row(p, r, h)],
                comb[h].at[:, pl.ds(0, DIM2)],
                sem_g[h],
            )
            pltpu.async_copy(t2_hbm.at[idx_row(p, r, h)], r2[h], sem_g[h])

        def drain(p, r, h):
            pltpu.make_async_copy(
                ta_hbm.at[idx_row(p, r, h)],
                comb[h].at[:, pl.ds(0, DIM2)],
                sem_g[h],
            ).wait()
            pltpu.make_async_copy(
                t2_hbm.at[idx_row(p, r, h)], r2[h], sem_g[h]
            ).wait()

        def stage_idx(g, p):
            pltpu.async_copy(wga_hbm.at[wid, g], idxa[p], sem_i[p])
            pltpu.async_copy(wgb_hbm.at[wid, g], idxb[p], sem_i[p])

        def drain_idx(g, p):
            pltpu.make_async_copy(wga_hbm.at[wid, g], idxa[p], sem_i[p]).wait()
            pltpu.make_async_copy(wgb_hbm.at[wid, g], idxb[p], sem_i[p]).wait()

        def proc(g, p, r, h, issue_next):
            size = HSIZE[h]
            drain(p, r, h)
            issue_next()

            def tail2(jj, c):
                for rr in range(2):
                    row = jj * 2 + rr
                    for k in range(DIM1 // LANES):
                        comb[h][row, pl.ds(DIM2 + k * LANES, LANES)] = r2[h][
                            row, pl.ds(DIM1 + k * LANES, LANES)
                        ]
                return c

            lax.fori_loop(0, size // 2, tail2, 0)
            pltpu.sync_copy(
                comb[h], out_hbm.at[base_row + g * RG + r, pl.ds(h * H0, size)]
            )

        def group(g, p, t, last_issue):
            for r in range(RG):
                for h in range(2):
                    if h == 0:
                        nxt = lambda p=p, r=r: issue(p, r, 1)
                    elif r < RG - 1:
                        nxt = lambda p=p, r=r: issue(p, r + 1, 0)
                    else:
                        nxt = last_issue
                    proc(g, p, r, h, nxt)

        def body(t, carry):
            g0 = 2 * t
            g1 = g0 + 1
            stage_idx(g1, 1)

            def into_g1():
                drain_idx(g1, 1)
                issue(1, 0, 0)

            group(g0, 0, t, into_g1)

            @pl.when(t < TBODY - 1)
            def _():
                stage_idx(g0 + 2, 0)

            def into_next_body():
                @pl.when(t < TBODY - 1)
                def _():
                    drain_idx(g0 + 2, 0)
                    issue(0, 0, 0)

            group(g1, 1, t, into_next_body)
            return carry

        pltpu.sync_copy(wga_hbm.at[wid, 0], idxa[0])
        pltpu.sync_copy(wgb_hbm.at[wid, 0], idxb[0])
        issue(0, 0, 0)
        lax.fori_loop(0, TBODY, body, 0)

    return stack_embed


_STACK_EMBED = _make_kernel()


def kernel(words, table1, table2):
    ta = jnp.concatenate([table1, table2[:, :DIM1]], axis=1)
    wr = words.reshape(NW, ROWS_PER_W, SEQ).astype(jnp.int32)
    wga = wr[:, :, :H0].reshape(NW, NGRP, RG, H0)
    wgb = wr[:, :, H0:].reshape(NW, NGRP, RG, H1)
    return _STACK_EMBED(wga, wgb, ta, table2)
